# baseline (speedup 1.0000x reference)
import functools

import jax
import jax.numpy as jnp
from jax import lax
from jax.experimental import pallas as pl
from jax.experimental.pallas import tpu as pltpu
from jax.experimental.pallas import tpu_sc as plsc

NUM_CRITERIA = 26
NUM_MODELS = 100000
D = 32
BATCH = 16384

NUM_CORES = 2
NUM_SUBCORES = 16
NUM_WORKERS = NUM_CORES * NUM_SUBCORES
B_PER_W = BATCH // NUM_WORKERS
LANES = 16

N_USEM = 16
VCHUNK = 128
N_VCHUNKS = B_PER_W // VCHUNK
VGROUPS = VCHUNK // LANES

_mesh = plsc.VectorSubcoreMesh(core_axis_name="c", subcore_axis_name="s")


@functools.partial(
    pl.kernel,
    out_type=jax.ShapeDtypeStruct((BATCH,), jnp.float32),
    mesh=_mesh,
    scratch_types=[
        pltpu.VMEM((B_PER_W,), jnp.int32),
        pltpu.VMEM((B_PER_W,), jnp.int32),
        pltpu.VMEM((B_PER_W,), jnp.int32),
        pltpu.VMEM((B_PER_W,), jnp.int32),
        pltpu.VMEM((B_PER_W,), jnp.int32),
        pltpu.VMEM((B_PER_W,), jnp.int32),
        pltpu.VMEM((B_PER_W // 4, 128), jnp.float32),
        pltpu.VMEM((2, VCHUNK, 128), jnp.float32),
        pltpu.VMEM((2, VCHUNK, 128), jnp.float32),
        pltpu.VMEM((B_PER_W,), jnp.float32),
        pltpu.VMEM((17 * LANES,), jnp.float32),
        pltpu.SemaphoreType.DMA,
        pltpu.SemaphoreType.DMA,
    ] + [pltpu.SemaphoreType.DMA] * N_USEM,
    compiler_params=pltpu.CompilerParams(
        needs_layout_passes=False, use_tc_tiling_on_sc=True),
)
def _vbt_kernel(c_hbm, i_hbm, j_hbm, k_hbm, u_hbm, vc_hbm, out_hbm,
                cidx_v, iidx_v, jc_v, jm_v, kc_v, km_v,
                ucomp, vj_rows, vk_rows, out_v, tbuf, sem_a, sem_b, *usems):
    wid = lax.axis_index("s") * NUM_CORES + lax.axis_index("c")
    base = wid * B_PER_W
    sems = (sem_a, sem_b)

    cp_c = pltpu.async_copy(c_hbm.at[pl.ds(base, B_PER_W)], cidx_v, sem_a)
    cp_i = pltpu.async_copy(i_hbm.at[pl.ds(base, B_PER_W)], iidx_v, sem_a)
    cp_j = pltpu.async_copy(j_hbm.at[pl.ds(base, B_PER_W)], jc_v, sem_a)
    cp_k = pltpu.async_copy(k_hbm.at[pl.ds(base, B_PER_W)], kc_v, sem_a)
    cp_c.wait()
    cp_i.wait()
    cp_j.wait()
    cp_k.wait()

    def idx_body(c, carry):
        sl = pl.ds(c * LANES, LANES)
        iidx_v[sl] = cidx_v[sl] * NUM_MODELS + iidx_v[sl]
        j = jc_v[sl]
        jc_v[sl] = j // 4
        jm_v[sl] = (j % 4) * 32
        k = kc_v[sl]
        kc_v[sl] = k // 4
        km_v[sl] = (k % 4) * 32
        return carry

    lax.fori_loop(0, B_PER_W // LANES, idx_body, 0)

    def u_fire(g, carry):
        fvec = iidx_v[pl.ds(g * LANES, LANES)]
        for e in range(LANES):
            line = g * (LANES // 4) + e // 4
            pltpu.async_copy(
                u_hbm.at[fvec[e]],
                ucomp.at[line, pl.ds((e % 4) * 32, 32)],
                usems[e % N_USEM if N_USEM <= LANES else e])
        return carry

    lax.fori_loop(0, B_PER_W // LANES, u_fire, 0)

    def v_fire(c, buf):
        sem = sems[buf]
        pltpu.async_copy(
            vc_hbm.at[jc_v.at[pl.ds(c * VCHUNK, VCHUNK)]],
            vj_rows.at[buf], sem)
        pltpu.async_copy(
            vc_hbm.at[kc_v.at[pl.ds(c * VCHUNK, VCHUNK)]],
            vk_rows.at[buf], sem)

    def v_drain(buf):
        sem = sems[buf]
        pltpu.make_async_copy(vc_hbm.at[pl.ds(0, VCHUNK)],
                              vj_rows.at[buf], sem).wait()
        pltpu.make_async_copy(vc_hbm.at[pl.ds(0, VCHUNK)],
                              vk_rows.at[buf], sem).wait()

    v_fire(0, 0)

    rows_per_sem = B_PER_W // N_USEM
    for s in range(N_USEM):
        pltpu.make_async_copy(
            vc_hbm.at[pl.ds(0, rows_per_sem // 4)],
            ucomp.at[pl.ds(s * (rows_per_sem // 4), rows_per_sem // 4)],
            usems[s]).wait()

    lane17 = lax.iota(jnp.int32, LANES) * 17
    one = jnp.full((LANES,), 1.0, jnp.float32)

    def compute(c, buf):
        def cbody(g, carry):
            jmvec = jm_v[pl.ds(c * VCHUNK + g * LANES, LANES)]
            kmvec = km_v[pl.ds(c * VCHUNK + g * LANES, LANES)]
            for e in range(LANES):
                slot = g * LANES + e
                b = c * VCHUNK + slot
                jm = jmvec[e]
                km = kmvec[e]
                u0 = ucomp[b // 4, pl.ds((e % 4) * 32, LANES)]
                u1 = ucomp[b // 4, pl.ds((e % 4) * 32 + LANES, LANES)]
                d0 = (vj_rows[buf, slot, pl.ds(jm, LANES)]
                      - vk_rows[buf, slot, pl.ds(km, LANES)])
                d1 = (vj_rows[buf, slot, pl.ds(jm + LANES, LANES)]
                      - vk_rows[buf, slot, pl.ds(km + LANES, LANES)])
                p = u0 * d0 + u1 * d1
                plsc.store_scatter(tbuf, [lane17 + e], p)
            acc = tbuf[pl.ds(0, LANES)]
            for l in range(1, LANES):
                acc = acc + tbuf[pl.ds(l * 17, LANES)]
            out_v[pl.ds(c * VCHUNK + g * LANES, LANES)] = (
                one / (one + jnp.exp(-acc)))
            return carry

        lax.fori_loop(0, VGROUPS, cbody, 0)

    for c in range(N_VCHUNKS):
        if c + 1 < N_VCHUNKS:
            v_fire(c + 1, (c + 1) % 2)
        v_drain(c % 2)
        compute(c, c % 2)

    pltpu.sync_copy(out_v, out_hbm.at[pl.ds(base, B_PER_W)])


@jax.jit
def kernel(criterion_idx, i_idx, j_idx, k_idx, u_weight, v_weight):
    vc = v_weight.reshape(NUM_MODELS // 4, 128)
    return _vbt_kernel(
        criterion_idx.astype(jnp.int32),
        i_idx.astype(jnp.int32),
        j_idx.astype(jnp.int32),
        k_idx.astype(jnp.int32),
        u_weight,
        vc,
    )


# speedup vs baseline: 1.0260x; 1.0260x over previous
import functools

import jax
import jax.numpy as jnp
from jax import lax
from jax.experimental import pallas as pl
from jax.experimental.pallas import tpu as pltpu
from jax.experimental.pallas import tpu_sc as plsc

NUM_CRITERIA = 26
NUM_MODELS = 100000
D = 32
BATCH = 16384

NUM_CORES = 2
NUM_SUBCORES = 16
NUM_WORKERS = NUM_CORES * NUM_SUBCORES
B_PER_W = BATCH // NUM_WORKERS
LANES = 16

N_USEM = 8
VCHUNK = 128
N_VCHUNKS = B_PER_W // VCHUNK
VGROUPS = VCHUNK // LANES

_mesh = plsc.VectorSubcoreMesh(core_axis_name="c", subcore_axis_name="s")


@functools.partial(
    pl.kernel,
    out_type=jax.ShapeDtypeStruct((BATCH,), jnp.float32),
    mesh=_mesh,
    scratch_types=[
        pltpu.VMEM((B_PER_W,), jnp.int32),
        pltpu.VMEM((B_PER_W,), jnp.int32),
        pltpu.VMEM((B_PER_W,), jnp.int32),
        pltpu.VMEM((B_PER_W,), jnp.int32),
        pltpu.VMEM((B_PER_W,), jnp.int32),
        pltpu.VMEM((B_PER_W,), jnp.int32),
        pltpu.VMEM((B_PER_W // 4, 128), jnp.float32),
        pltpu.VMEM((2, VCHUNK, 128), jnp.float32),
        pltpu.VMEM((2, VCHUNK, 128), jnp.float32),
        pltpu.VMEM((B_PER_W,), jnp.float32),
        pltpu.VMEM((17 * LANES,), jnp.float32),
        pltpu.SemaphoreType.DMA,
        pltpu.SemaphoreType.DMA,
    ] + [pltpu.SemaphoreType.DMA] * N_USEM,
    compiler_params=pltpu.CompilerParams(
        needs_layout_passes=False, use_tc_tiling_on_sc=True),
)
def _vbt_kernel(c_hbm, i_hbm, j_hbm, k_hbm, u_hbm, vc_hbm, out_hbm,
                cidx_v, iidx_v, jc_v, jm_v, kc_v, km_v,
                ucomp, vj_rows, vk_rows, out_v, tbuf, sem_a, sem_b, *usems):
    wid = lax.axis_index("s") * NUM_CORES + lax.axis_index("c")
    base = wid * B_PER_W
    sems = (sem_a, sem_b)

    cp_c = pltpu.async_copy(c_hbm.at[pl.ds(base, B_PER_W)], cidx_v, sem_a)
    cp_i = pltpu.async_copy(i_hbm.at[pl.ds(base, B_PER_W)], iidx_v, sem_a)
    cp_j = pltpu.async_copy(j_hbm.at[pl.ds(base, B_PER_W)], jc_v, sem_a)
    cp_k = pltpu.async_copy(k_hbm.at[pl.ds(base, B_PER_W)], kc_v, sem_a)
    cp_c.wait()
    cp_i.wait()
    cp_j.wait()
    cp_k.wait()

    def idx_body(c, carry):
        sl = pl.ds(c * LANES, LANES)
        iidx_v[sl] = cidx_v[sl] * NUM_MODELS + iidx_v[sl]
        j = jc_v[sl]
        jc_v[sl] = j // 4
        jm_v[sl] = (j % 4) * 32
        k = kc_v[sl]
        kc_v[sl] = k // 4
        km_v[sl] = (k % 4) * 32
        return carry

    lax.fori_loop(0, B_PER_W // LANES, idx_body, 0)

    def u_fire(g, carry):
        fvec = iidx_v[pl.ds(g * LANES, LANES)]
        for e in range(LANES):
            line = g * (LANES // 4) + e // 4
            pltpu.async_copy(
                u_hbm.at[fvec[e]],
                ucomp.at[line, pl.ds((e % 4) * 32, 32)],
                usems[e % N_USEM])
        return carry

    lax.fori_loop(0, B_PER_W // LANES, u_fire, 0)

    def v_fire(c, buf):
        sem = sems[buf]
        pltpu.async_copy(
            vc_hbm.at[jc_v.at[pl.ds(c * VCHUNK, VCHUNK)]],
            vj_rows.at[buf], sem)
        pltpu.async_copy(
            vc_hbm.at[kc_v.at[pl.ds(c * VCHUNK, VCHUNK)]],
            vk_rows.at[buf], sem)

    def v_drain(buf):
        sem = sems[buf]
        pltpu.make_async_copy(vc_hbm.at[pl.ds(0, VCHUNK)],
                              vj_rows.at[buf], sem).wait()
        pltpu.make_async_copy(vc_hbm.at[pl.ds(0, VCHUNK)],
                              vk_rows.at[buf], sem).wait()

    v_fire(0, 0)

    lines_per_sem = (B_PER_W // N_USEM) // 4
    for s in range(N_USEM):
        pltpu.make_async_copy(
            vc_hbm.at[pl.ds(0, lines_per_sem)],
            ucomp.at[pl.ds(s * lines_per_sem, lines_per_sem)],
            usems[s]).wait()

    lane17 = lax.iota(jnp.int32, LANES) * 17
    one = jnp.full((LANES,), 1.0, jnp.float32)

    def compute(c, buf):
        def cbody(g, carry):
            jmvec = jm_v[pl.ds(c * VCHUNK + g * LANES, LANES)]
            kmvec = km_v[pl.ds(c * VCHUNK + g * LANES, LANES)]
            for e in range(LANES):
                slot = g * LANES + e
                b = c * VCHUNK + slot
                jm = jmvec[e]
                km = kmvec[e]
                u0 = ucomp[b // 4, pl.ds((e % 4) * 32, LANES)]
                u1 = ucomp[b // 4, pl.ds((e % 4) * 32 + LANES, LANES)]
                d0 = (vj_rows[buf, slot, pl.ds(jm, LANES)]
                      - vk_rows[buf, slot, pl.ds(km, LANES)])
                d1 = (vj_rows[buf, slot, pl.ds(jm + LANES, LANES)]
                      - vk_rows[buf, slot, pl.ds(km + LANES, LANES)])
                p = u0 * d0 + u1 * d1
                plsc.store_scatter(tbuf, [lane17 + e], p)
            acc = tbuf[pl.ds(0, LANES)]
            for l in range(1, LANES):
                acc = acc + tbuf[pl.ds(l * 17, LANES)]
            out_v[pl.ds(c * VCHUNK + g * LANES, LANES)] = (
                one / (one + jnp.exp(-acc)))
            return carry

        lax.fori_loop(0, VGROUPS, cbody, 0)

    for c in range(N_VCHUNKS):
        if c + 1 < N_VCHUNKS:
            v_fire(c + 1, (c + 1) % 2)
        v_drain(c % 2)
        compute(c, c % 2)

    pltpu.sync_copy(out_v, out_hbm.at[pl.ds(base, B_PER_W)])


@jax.jit
def kernel(criterion_idx, i_idx, j_idx, k_idx, u_weight, v_weight):
    vc = v_weight.reshape(NUM_MODELS // 4, 128)
    out = _vbt_kernel(
        criterion_idx.astype(jnp.int32),
        i_idx.astype(jnp.int32),
        j_idx.astype(jnp.int32),
        k_idx.astype(jnp.int32),
        u_weight,
        vc,
    )
    return out
